# Initial kernel scaffold; baseline (speedup 1.0000x reference)
import jax
import jax.numpy as jnp
from jax.experimental import pallas as pl


def kernel(atom_fea, crystal_atom_idx, W1, b1, W2, b2, Wp, bp):
    raise NotImplementedError("write your pallas kernel here")


# speedup vs baseline: 2.4928x; 2.4928x over previous
import functools

import jax
import jax.numpy as jnp
from jax import lax
from jax.experimental import pallas as pl
from jax.experimental.pallas import tpu as pltpu
from jax.experimental.pallas import tpu_sc as plsc


HP = 16


def _row_block(n, cap=8192):
    best = 8
    for r in range(8, min(n, cap) + 1, 8):
        if n % r == 0:
            best = r
    return best


def _mlp_weights(atom_fea, W1, b1_2d, W2p, b2p_2d):
    N, D = atom_fea.shape
    HID = W1.shape[1]
    R = _row_block(N)

    def body(x_ref, w1_ref, b1_ref, w2_ref, b2_ref, o_ref):
        x = x_ref[...]
        h1 = jnp.dot(x, w1_ref[...], preferred_element_type=jnp.float32)
        h1 = h1 + b1_ref[...]
        h1 = h1 * (1.0 / (1.0 + jnp.exp(-h1)))
        lg = jnp.dot(h1, w2_ref[...], preferred_element_type=jnp.float32)
        lg = lg + b2_ref[...]
        m = jnp.max(lg, axis=-1, keepdims=True)
        e = jnp.exp(lg - m)
        o_ref[...] = e / jnp.sum(e, axis=-1, keepdims=True)

    return pl.pallas_call(
        body,
        grid=(N // R,),
        in_specs=[
            pl.BlockSpec((R, D), lambda i: (i, 0)),
            pl.BlockSpec((D, HID), lambda i: (0, 0)),
            pl.BlockSpec((1, HID), lambda i: (0, 0)),
            pl.BlockSpec((HID, HP), lambda i: (0, 0)),
            pl.BlockSpec((1, HP), lambda i: (0, 0)),
        ],
        out_specs=pl.BlockSpec((R, HP), lambda i: (i, 0)),
        out_shape=jax.ShapeDtypeStruct((N, HP), jnp.float32),
    )(atom_fea, W1, b1_2d, W2p, b2p_2d)


def _sc_pool(atom_fea, cw, idx, H):
    N, D = atom_fea.shape
    B, A = idx.shape
    info = plsc.get_sparse_core_info()
    NC, NS, L = info.num_cores, info.num_subcores, info.num_lanes
    NW = NC * NS
    per_w = B // NW
    nseg = D // L
    mesh = plsc.VectorSubcoreMesh(core_axis_name="c", subcore_axis_name="s")

    @functools.partial(
        pl.kernel,
        mesh=mesh,
        compiler_params=pltpu.CompilerParams(use_tc_tiling_on_sc=False),
        out_type=jax.ShapeDtypeStruct((B, H * D), jnp.float32),
        scratch_types=[
            pltpu.VMEM((per_w, A), jnp.int32),
            pltpu.VMEM((A, D), jnp.float32),
            pltpu.VMEM((A, HP), jnp.float32),
            pltpu.VMEM((H * D,), jnp.float32),
            pltpu.SemaphoreType.DMA,
            pltpu.SemaphoreType.DMA,
        ],
    )
    def pool(atom_hbm, cw_hbm, idx_hbm, out_hbm,
             idx_v, rows_v, cwr_v, out_v, sem_a, sem_w):
        wid = lax.axis_index("s") * NC + lax.axis_index("c")
        base = wid * per_w
        pltpu.sync_copy(idx_hbm.at[pl.ds(base, per_w)], idx_v)

        def crystal(i, carry):
            irow = idx_v.at[i]
            cp_a = pltpu.async_copy(atom_hbm.at[irow], rows_v, sem_a)
            cp_w = pltpu.async_copy(cw_hbm.at[irow], cwr_v, sem_w)
            cp_a.wait()
            cp_w.wait()

            def atom(a, accs):
                accs = list(accs)
                cwvec = cwr_v[a, :]
                dnums = lax.GatherDimensionNumbers(
                    offset_dims=(), collapsed_slice_dims=(0,),
                    start_index_map=(0,))
                cwb = [
                    lax.gather(cwvec, jnp.full((L, 1), h, jnp.int32), dnums,
                               slice_sizes=(1,),
                               mode=lax.GatherScatterMode.PROMISE_IN_BOUNDS)
                    for h in range(H)
                ]
                for seg in range(nseg):
                    v = rows_v[a, pl.ds(seg * L, L)]
                    for h in range(H):
                        accs[h * nseg + seg] = accs[h * nseg + seg] + cwb[h] * v
                return tuple(accs)

            accs = lax.fori_loop(
                0, A, atom,
                tuple(jnp.zeros((L,), jnp.float32) for _ in range(H * nseg)))
            for j in range(H * nseg):
                out_v[pl.ds(j * L, L)] = accs[j]
            pltpu.sync_copy(out_v, out_hbm.at[base + i])
            return carry

        lax.fori_loop(0, per_w, crystal, 0)

    return pool(atom_fea, cw, idx)


def _project(flat, Wp, bp_2d):
    B, HD = flat.shape
    D = Wp.shape[1]

    def body(f_ref, wp_ref, bp_ref, o_ref):
        y = jnp.dot(f_ref[...], wp_ref[...], preferred_element_type=jnp.float32)
        y = y + bp_ref[...]
        o_ref[...] = y * (1.0 / (1.0 + jnp.exp(-y)))

    return pl.pallas_call(
        body,
        out_shape=jax.ShapeDtypeStruct((B, D), jnp.float32),
    )(flat, Wp, bp_2d)


def kernel(atom_fea, crystal_atom_idx, W1, b1, W2, b2, Wp, bp):
    H = W2.shape[1]
    W2p = jnp.pad(W2, ((0, 0), (0, HP - H)))
    b2p = jnp.concatenate([b2, jnp.full((HP - H,), -1e30, b2.dtype)])
    cw = _mlp_weights(atom_fea, W1, b1.reshape(1, -1), W2p, b2p.reshape(1, -1))
    flat = _sc_pool(atom_fea, cw, crystal_atom_idx, H)
    return _project(flat, Wp, bp.reshape(1, -1))
